# Initial kernel scaffold; baseline (speedup 1.0000x reference)
import jax
import jax.numpy as jnp
from jax.experimental import pallas as pl


def kernel(x, W_enc, b_enc, W_dec, b_dec):
    raise NotImplementedError("write your pallas kernel here")


# speedup vs baseline: 8.4078x; 8.4078x over previous
import functools

import jax
import jax.numpy as jnp
from jax.experimental import pallas as pl
from jax.experimental.pallas import tpu as pltpu

K_TOP = 128


def _enc_kernel(x_ref, w_ref, b_ref, z_ref, *, n_hid, bh, br, hidden):
    j = pl.program_id(1)
    pre = jnp.dot(x_ref[...], w_ref[...], preferred_element_type=jnp.float32)
    pre = jnp.maximum(pre + b_ref[...], 0.0)
    z_ref[:, pl.ds(j * bh, bh)] = pre

    @pl.when(j == n_hid - 1)
    def _select_and_mask():
        chk = bh
        n_chk = hidden // chk

        def count_ge(cand):
            def cbody(c, acc):
                blk = z_ref[:, pl.ds(c * chk, chk)]
                bits = jax.lax.bitcast_convert_type(blk, jnp.int32)
                return acc + jnp.sum((bits >= cand).astype(jnp.int32), axis=1,
                                     keepdims=True)
            return jax.lax.fori_loop(0, n_chk, cbody,
                                     jnp.zeros((br, 1), jnp.int32))

        def bbody(i, t):
            b = 30 - i
            cand = t | jnp.left_shift(1, b)
            cnt = count_ge(cand)
            return jnp.where(cnt >= K_TOP, cand, t)

        t = jax.lax.fori_loop(0, 31, bbody, jnp.zeros((br, 1), jnp.int32))

        def mbody(c, _):
            blk = z_ref[:, pl.ds(c * chk, chk)]
            bits = jax.lax.bitcast_convert_type(blk, jnp.int32)
            z_ref[:, pl.ds(c * chk, chk)] = jnp.where(bits >= t, blk, 0.0)
            return 0
        jax.lax.fori_loop(0, n_chk, mbody, 0)


def _dec_kernel(z_ref, w_ref, b_ref, o_ref):
    j = pl.program_id(1)

    @pl.when(j == 0)
    def _init():
        o_ref[...] = jnp.broadcast_to(b_ref[...], o_ref.shape)

    o_ref[...] += jnp.dot(z_ref[...], w_ref[...],
                          preferred_element_type=jnp.float32)


@jax.jit
def kernel(x, W_enc, b_enc, W_dec, b_dec):
    n, d_in = x.shape
    hidden = W_enc.shape[1]

    br = min(256, n)
    bh = min(512, hidden)
    n_hid = hidden // bh

    z = pl.pallas_call(
        functools.partial(_enc_kernel, n_hid=n_hid, bh=bh, br=br,
                          hidden=hidden),
        grid=(n // br, n_hid),
        in_specs=[
            pl.BlockSpec((br, d_in), lambda i, j: (i, 0)),
            pl.BlockSpec((d_in, bh), lambda i, j: (0, j)),
            pl.BlockSpec((1, bh), lambda i, j: (0, j)),
        ],
        out_specs=pl.BlockSpec((br, hidden), lambda i, j: (i, 0)),
        out_shape=jax.ShapeDtypeStruct((n, hidden), jnp.float32),
        compiler_params=pltpu.CompilerParams(
            dimension_semantics=("parallel", "arbitrary")),
    )(x, W_enc, b_enc.reshape(1, hidden))

    br2 = min(1024, n)
    bh2 = min(512, hidden)
    x_recon = pl.pallas_call(
        _dec_kernel,
        grid=(n // br2, hidden // bh2),
        in_specs=[
            pl.BlockSpec((br2, bh2), lambda i, j: (i, j)),
            pl.BlockSpec((bh2, d_in), lambda i, j: (j, 0)),
            pl.BlockSpec((1, d_in), lambda i, j: (0, 0)),
        ],
        out_specs=pl.BlockSpec((br2, d_in), lambda i, j: (i, 0)),
        out_shape=jax.ShapeDtypeStruct((n, d_in), jnp.float32),
        compiler_params=pltpu.CompilerParams(
            dimension_semantics=("parallel", "arbitrary")),
    )(z, W_dec, b_dec.reshape(1, d_in))

    return (z, x_recon)
